# baseline (speedup 1.0000x reference)
import functools

import jax
import jax.numpy as jnp
from jax.experimental import pallas as pl

D = 64


def _edge_msg_kernel(gd_ref, gs_ref, e_ref, w_ref, b_ref, out_ref):
    fs = (
        gd_ref[...]
        + gs_ref[...]
        + jnp.dot(e_ref[...], w_ref[...], preferred_element_type=jnp.float32)
        + b_ref[...]
    )
    f = fs[:, :D]
    s = fs[:, D:]
    sp = jnp.maximum(s, 0.0) + jnp.log1p(jnp.exp(-jnp.abs(s)))
    out_ref[...] = (1.0 / (1.0 + jnp.exp(-f))) * sp


def _edge_messages(gd, gs, e, ce, cb):
    E = gd.shape[0]
    BE = 1024
    grid = (pl.cdiv(E, BE),)
    return pl.pallas_call(
        _edge_msg_kernel,
        grid=grid,
        in_specs=[
            pl.BlockSpec((BE, 2 * D), lambda i: (i, 0)),
            pl.BlockSpec((BE, 2 * D), lambda i: (i, 0)),
            pl.BlockSpec((BE, D), lambda i: (i, 0)),
            pl.BlockSpec((D, 2 * D), lambda i: (0, 0)),
            pl.BlockSpec((1, 2 * D), lambda i: (0, 0)),
        ],
        out_specs=pl.BlockSpec((BE, D), lambda i: (i, 0)),
        out_shape=jax.ShapeDtypeStruct((E, D), jnp.float32),
    )(gd, gs, e, ce, cb)


def kernel(x, edge_index, edge_attr, batch, W_emb1, b_emb1, W_emb2, b_emb2,
           Wf0, bf0, Ws0, bs0, gamma0, beta0, Wf1, bf1, Ws1, bs1, gamma1,
           beta1, Wf2, bf2, Ws2, bs2, gamma2, beta2, W_l1, b_l1, W_l2, b_l2,
           W_out, b_out):
    N = x.shape[0]
    G = 16
    src = edge_index[0]
    dst = edge_index[1]

    h = x @ W_emb1.T + b_emb1
    e = edge_attr @ W_emb2.T + b_emb2

    convs = [
        (Wf0, bf0, Ws0, bs0, gamma0, beta0),
        (Wf1, bf1, Ws1, bs1, gamma1, beta1),
        (Wf2, bf2, Ws2, bs2, gamma2, beta2),
    ]
    n_conv = len(convs)
    for i, (Wf, bf, Ws, bs, gamma, beta) in enumerate(convs):
        wp = jnp.concatenate([Wf[:, :D].T, Ws[:, :D].T], axis=1)
        wq = jnp.concatenate([Wf[:, D:2 * D].T, Ws[:, D:2 * D].T], axis=1)
        ce = jnp.concatenate([Wf[:, 2 * D:].T, Ws[:, 2 * D:].T], axis=1)
        cb = jnp.concatenate([bf, bs])[None, :]
        P = h @ wp
        Q = h @ wq
        gd = P[dst]
        gs = Q[src]
        msg = _edge_messages(gd, gs, e, ce, cb)
        agg = jax.ops.segment_sum(msg, dst, num_segments=N)
        h = h + agg
        mu = jnp.mean(h, axis=0, keepdims=True)
        var = jnp.var(h, axis=0, keepdims=True)
        h = (h - mu) / jnp.sqrt(var + 1e-5) * gamma + beta
        if i < n_conv - 1:
            h = jax.nn.relu(h)

    sums = jax.ops.segment_sum(h, batch, num_segments=G)
    cnt = jax.ops.segment_sum(jnp.ones((N, 1), jnp.float32), batch,
                              num_segments=G)
    y = sums / jnp.maximum(cnt, 1.0)
    y = jax.nn.softplus(y @ W_l1.T + b_l1)
    y = jax.nn.softplus(y @ W_l2.T + b_l2)
    y = (y @ W_out.T + b_out).squeeze(-1)
    return y


# speedup vs baseline: 1.7109x; 1.7109x over previous
import functools

import jax
import jax.numpy as jnp
from jax import lax
from jax.experimental import pallas as pl
from jax.experimental.pallas import tpu as pltpu
from jax.experimental.pallas import tpu_sc as plsc

D = 64
NC = 2
NS = 16
NW = NC * NS
CG = 128
CS = 64

_mesh = functools.partial(
    plsc.VectorSubcoreMesh, core_axis_name="c", subcore_axis_name="s")


def _gather_body(n_chunks, n_it, p_hbm, q_hbm, dst_hbm, src_hbm, gd_hbm,
                 gs_hbm, idxd0, idxd1, idxs0, idxs1, bufa0, bufa1, bufb0,
                 bufb1, sa0, sa1, sb0, sb1):
    wid = lax.axis_index("c") * NS + lax.axis_index("s")
    idxd = (idxd0, idxd1)
    idxs = (idxs0, idxs1)
    bufa = (bufa0, bufa1)
    bufb = (bufb0, bufb1)
    sa = (sa0, sa1)
    sb = (sb0, sb1)

    def start(it, sl):
        cid = wid + it * NW
        @pl.when((it < n_it) & (cid < n_chunks))
        def _():
            b = cid * CG
            pltpu.sync_copy(dst_hbm.at[pl.ds(b, CG)], idxd[sl])
            pltpu.sync_copy(src_hbm.at[pl.ds(b, CG)], idxs[sl])
            pltpu.async_copy(p_hbm.at[idxd[sl]], bufa[sl], sa[sl])
            pltpu.async_copy(q_hbm.at[idxs[sl]], bufb[sl], sb[sl])

    def finish(it, sl):
        cid = wid + it * NW
        @pl.when(cid < n_chunks)
        def _():
            pltpu.make_async_copy(p_hbm.at[idxd[sl]], bufa[sl], sa[sl]).wait()
            pltpu.make_async_copy(q_hbm.at[idxs[sl]], bufb[sl], sb[sl]).wait()
            b = cid * CG
            pltpu.sync_copy(bufa[sl], gd_hbm.at[pl.ds(b, CG)])
            pltpu.sync_copy(bufb[sl], gs_hbm.at[pl.ds(b, CG)])

    start(0, 0)

    def body(jj, carry):
        it = 2 * jj
        start(it + 1, 1)
        finish(it, 0)
        start(it + 2, 0)
        finish(it + 1, 1)
        return carry

    lax.fori_loop(0, n_it // 2, body, 0, unroll=False)


def _sc_gather(p, q, dst, src):
    e = dst.shape[0]
    assert e % CG == 0
    n_chunks = e // CG
    n_it = -(-n_chunks // NW)
    n_it += n_it % 2
    f = pl.kernel(
        functools.partial(_gather_body, n_chunks, n_it),
        out_type=(jax.ShapeDtypeStruct((e, 2 * D), jnp.float32),
                  jax.ShapeDtypeStruct((e, 2 * D), jnp.float32)),
        mesh=_mesh(),
        scratch_types=(
            [pltpu.VMEM((CG,), jnp.int32) for _ in range(4)]
            + [pltpu.VMEM((CG, 2 * D), jnp.float32) for _ in range(4)]
            + [pltpu.SemaphoreType.DMA for _ in range(4)]
        ),
    )
    return f(p, q, dst, src)


def _scatter_body(n_chunks, n_it, nhalf, dst_hbm, msg_hbm, out_hbm, idx0,
                  idx1, lidx0, lidx1, msg0, msg1, acc, sm0, sm1):
    cidx = lax.axis_index("c")
    sid = lax.axis_index("s")
    base = cidx * nhalf
    idx = (idx0, idx1)
    lidx = (lidx0, lidx1)
    msg = (msg0, msg1)
    sm = (sm0, sm1)
    stripe = nhalf // NS
    n_full = stripe // CS
    rem = stripe - n_full * CS

    def zrow(i, carry):
        for c4 in range(4):
            msg0[i, pl.ds(c4 * 16, 16)] = jnp.zeros((16,), jnp.float32)
        return carry
    lax.fori_loop(0, CS, zrow, 0, unroll=False)
    for part in range(n_full):
        pltpu.sync_copy(
            msg0, acc.at[pl.ds(sid * stripe + part * CS, CS)])
    if rem:
        pltpu.sync_copy(
            msg0.at[pl.ds(0, rem)],
            acc.at[pl.ds(sid * stripe + n_full * CS, rem)])
    plsc.subcore_barrier()

    def body(it, carry):
        cid = sid + it * NS
        @pl.when(cid < n_chunks)
        def _():
            b = cid * CS
            pltpu.sync_copy(dst_hbm.at[pl.ds(b, CS)], idx0)
            pltpu.sync_copy(msg_hbm.at[pl.ds(b, CS)], msg1)
            for j in range(CS // 16):
                v = idx0[pl.ds(j * 16, 16)] - base
                ok = (v >= 0) & (v < nhalf)
                lidx0[pl.ds(j * 16, 16)] = jnp.where(ok, v, nhalf)
            pltpu.sync_copy(msg1, acc.at[lidx0], add=True)
        return carry

    lax.fori_loop(0, n_it, body, 0, unroll=False)
    plsc.subcore_barrier()

    for part in range(n_full):
        r0 = sid * stripe + part * CS
        pltpu.sync_copy(acc.at[pl.ds(r0, CS)], msg0)
        pltpu.sync_copy(msg0, out_hbm.at[pl.ds(base + r0, CS)])
    if rem:
        r0 = sid * stripe + n_full * CS
        pltpu.sync_copy(acc.at[pl.ds(r0, rem)], msg0.at[pl.ds(0, rem)])
        pltpu.sync_copy(msg0.at[pl.ds(0, rem)],
                        out_hbm.at[pl.ds(base + r0, rem)])


def _sc_segment_sum(msg, dst, n):
    e = msg.shape[0]
    assert e % CS == 0
    n_chunks = e // CS
    n_it = -(-n_chunks // NS)
    n_it += n_it % 2
    nhalf = ((n + 1) // 2 + NS * 32 - 1) // (NS * 32) * (NS * 32)
    npad = 2 * nhalf
    stripe = nhalf // NS
    f = pl.kernel(
        functools.partial(_scatter_body, n_chunks, n_it, nhalf),
        out_type=jax.ShapeDtypeStruct((npad, D), jnp.float32),
        mesh=_mesh(),
        scratch_types=(
            [pltpu.VMEM((CS,), jnp.int32) for _ in range(4)]
            + [pltpu.VMEM((CS, D), jnp.float32) for _ in range(2)]
            + [pltpu.VMEM_SHARED((nhalf + 8, D), jnp.float32)]
            + [pltpu.SemaphoreType.DMA for _ in range(2)]
        ),
    )
    return f(dst, msg)[:n]


def _edge_msg_kernel(gd_ref, gs_ref, e_ref, w_ref, b_ref, out_ref):
    fs = (
        gd_ref[...]
        + gs_ref[...]
        + jnp.dot(e_ref[...], w_ref[...], preferred_element_type=jnp.float32)
        + b_ref[...]
    )
    f = fs[:, :D]
    s = fs[:, D:]
    sp = jnp.maximum(s, 0.0) + jnp.log1p(jnp.exp(-jnp.abs(s)))
    out_ref[...] = (1.0 / (1.0 + jnp.exp(-f))) * sp


def _edge_messages(gd, gs, e, ce, cb):
    E = gd.shape[0]
    BE = 1024
    grid = (pl.cdiv(E, BE),)
    return pl.pallas_call(
        _edge_msg_kernel,
        grid=grid,
        in_specs=[
            pl.BlockSpec((BE, 2 * D), lambda i: (i, 0)),
            pl.BlockSpec((BE, 2 * D), lambda i: (i, 0)),
            pl.BlockSpec((BE, D), lambda i: (i, 0)),
            pl.BlockSpec((D, 2 * D), lambda i: (0, 0)),
            pl.BlockSpec((1, 2 * D), lambda i: (0, 0)),
        ],
        out_specs=pl.BlockSpec((BE, D), lambda i: (i, 0)),
        out_shape=jax.ShapeDtypeStruct((E, D), jnp.float32),
    )(gd, gs, e, ce, cb)


def kernel(x, edge_index, edge_attr, batch, W_emb1, b_emb1, W_emb2, b_emb2,
           Wf0, bf0, Ws0, bs0, gamma0, beta0, Wf1, bf1, Ws1, bs1, gamma1,
           beta1, Wf2, bf2, Ws2, bs2, gamma2, beta2, W_l1, b_l1, W_l2, b_l2,
           W_out, b_out):
    N = x.shape[0]
    G = 16
    src = edge_index[0]
    dst = edge_index[1]

    h = x @ W_emb1.T + b_emb1
    e = edge_attr @ W_emb2.T + b_emb2

    convs = [
        (Wf0, bf0, Ws0, bs0, gamma0, beta0),
        (Wf1, bf1, Ws1, bs1, gamma1, beta1),
        (Wf2, bf2, Ws2, bs2, gamma2, beta2),
    ]
    n_conv = len(convs)
    for i, (Wf, bf, Ws, bs, gamma, beta) in enumerate(convs):
        wp = jnp.concatenate([Wf[:, :D].T, Ws[:, :D].T], axis=1)
        wq = jnp.concatenate([Wf[:, D:2 * D].T, Ws[:, D:2 * D].T], axis=1)
        ce = jnp.concatenate([Wf[:, 2 * D:].T, Ws[:, 2 * D:].T], axis=1)
        cb = jnp.concatenate([bf, bs])[None, :]
        P = h @ wp
        Q = h @ wq
        gd, gs = _sc_gather(P, Q, dst, src)
        msg = _edge_messages(gd, gs, e, ce, cb)
        agg = jax.ops.segment_sum(msg, dst, num_segments=N)
        h = h + agg
        mu = jnp.mean(h, axis=0, keepdims=True)
        var = jnp.var(h, axis=0, keepdims=True)
        h = (h - mu) / jnp.sqrt(var + 1e-5) * gamma + beta
        if i < n_conv - 1:
            h = jax.nn.relu(h)

    sums = jax.ops.segment_sum(h, batch, num_segments=G)
    cnt = jax.ops.segment_sum(jnp.ones((N, 1), jnp.float32), batch,
                              num_segments=G)
    y = sums / jnp.maximum(cnt, 1.0)
    y = jax.nn.softplus(y @ W_l1.T + b_l1)
    y = jax.nn.softplus(y @ W_l2.T + b_l2)
    y = (y @ W_out.T + b_out).squeeze(-1)
    return y
